# Initial kernel scaffold; baseline (speedup 1.0000x reference)
import jax
import jax.numpy as jnp
from jax.experimental import pallas as pl


def kernel(nodeInfosTensor, parent_index, W_resize, b_resize, W_parent, b_parent, W_nbr, b_nbr, W_ih, W_hh, b_ih, b_hh):
    raise NotImplementedError("write your pallas kernel here")


# speedup vs baseline: 799.3600x; 799.3600x over previous
import functools

import jax
import jax.numpy as jnp
from jax import lax
from jax.experimental import pallas as pl
from jax.experimental.pallas import tpu as pltpu
from jax.experimental.pallas import tpu_sc as plsc

_B, _N, _D = 4, 10000, 128
_NP = 10240
_R = _B * _NP
_NW = 32
_PER_W = _R // _NW
_PER_S = _NP // 16
_BLK = 512
_NB = _R // _BLK


def _resize_body(x_ref, wr_ref, br_ref, h_ref):
    h_ref[...] = jnp.dot(x_ref[...], wr_ref[...],
                         preferred_element_type=jnp.float32) + br_ref[...]


def _cat_body(h_ref, wcat_ref, bcat_ref, hp_ref, hn_ref, gi_ref):
    cat = jnp.dot(h_ref[...], wcat_ref[...],
                  preferred_element_type=jnp.float32) + bcat_ref[...]
    hp_ref[...] = cat[:, :128]
    hn_ref[...] = cat[:, 128:256]
    gi_ref[...] = cat[:, 256:]


def _gru_body(p_ref, s_ref, c_ref, gi_ref, whh_ref, bhh_ref, bnbr_ref,
              h_ref):
    inv = 1.0 / jnp.maximum(c_ref[...][:, :1], 1.0)
    summary = p_ref[...] + s_ref[...] * inv + bnbr_ref[...]
    gh = jnp.dot(summary, whh_ref[...],
                 preferred_element_type=jnp.float32) + bhh_ref[...]
    gi = gi_ref[...]
    r = jax.nn.sigmoid(gi[:, :128] + gh[:, :128])
    z = jax.nn.sigmoid(gi[:, 128:256] + gh[:, 128:256])
    n = jnp.tanh(gi[:, 256:] + r * gh[:, 256:])
    h_ref[...] = (1.0 - z) * n + z * summary


def _row_spec(width):
    return pl.BlockSpec((_BLK, width), lambda i: (i, 0))


def _whole(shape):
    return pl.BlockSpec(shape, lambda i: tuple(0 for _ in shape))


_H_SHAPE = jax.ShapeDtypeStruct((_R, 128), jnp.float32)


def _stage_resize(xf, wr, br):
    return pl.pallas_call(
        _resize_body,
        grid=(_NB,),
        in_specs=[_row_spec(128), _whole((128, 128)), _whole((1, 128))],
        out_specs=_row_spec(128),
        out_shape=_H_SHAPE,
    )(xf, wr, br)


def _stage_cat(h, wcat, bcat):
    return pl.pallas_call(
        _cat_body,
        grid=(_NB,),
        in_specs=[_row_spec(128), _whole((128, 640)), _whole((1, 640))],
        out_specs=[_row_spec(128), _row_spec(128), _row_spec(384)],
        out_shape=[_H_SHAPE, _H_SHAPE,
                   jax.ShapeDtypeStruct((_R, 384), jnp.float32)],
    )(h, wcat, bcat)


def _stage_gru(p, s, cnt, gi, whh, bhh, bnbr):
    return pl.pallas_call(
        _gru_body,
        grid=(_NB,),
        in_specs=[_row_spec(128), _row_spec(128), _row_spec(128),
                  _row_spec(384), _whole((128, 384)), _whole((1, 384)),
                  _whole((1, 128))],
        out_specs=_row_spec(128),
        out_shape=_H_SHAPE,
    )(p, s, cnt, gi, whh, bhh, bnbr)


def _route_body(hp, hn, gidxf, sidxf, zrow, onerow,
                p_out, s_out, c_out,
                rows, gidxv, sidxv,
                acc, sem):
    c = lax.axis_index("c")
    s = lax.axis_index("s")
    w = c * 16 + s
    pltpu.sync_copy(gidxf.at[pl.ds(w * 16, 16)], gidxv)
    for k in range(10):
        pltpu.async_copy(hp.at[gidxv.at[k]], rows, sem).wait()
        pltpu.sync_copy(rows, p_out.at[pl.ds(w * _PER_W + k * 128, 128)])
    for tr in range(2):
        t = 2 * c + tr
        pltpu.sync_copy(sidxf.at[pl.ds((t * 16 + s) * 8, 8)], sidxv)
        for use_ones in (False, True):
            out = c_out if use_ones else s_out
            pltpu.sync_copy(zrow, rows)
            for i in range(5):
                pltpu.sync_copy(
                    rows, acc.at[pl.ds(s * _PER_S + i * 128, 128)])
            plsc.subcore_barrier()
            if use_ones:
                pltpu.sync_copy(onerow, rows)
                for k in range(5):
                    pltpu.sync_copy(rows, acc.at[sidxv.at[k]], add=True)
            else:
                for k in range(5):
                    pltpu.sync_copy(
                        hn.at[pl.ds(t * _NP + s * _PER_S + k * 128, 128)],
                        rows)
                    pltpu.sync_copy(rows, acc.at[sidxv.at[k]], add=True)
            plsc.subcore_barrier()
            for i in range(5):
                pltpu.sync_copy(
                    acc.at[pl.ds(s * _PER_S + i * 128, 128)], rows)
                pltpu.sync_copy(
                    rows,
                    out.at[pl.ds(t * _NP + s * _PER_S + i * 128, 128)])


@functools.lru_cache(maxsize=None)
def _get_route():
    mesh = plsc.VectorSubcoreMesh(core_axis_name="c", subcore_axis_name="s")
    outs = [jax.ShapeDtypeStruct((_R, 128), jnp.float32),
            jax.ShapeDtypeStruct((_R, 128), jnp.float32),
            jax.ShapeDtypeStruct((_R, 128), jnp.float32)]
    scratch = [pltpu.VMEM((128, 128), jnp.float32),
               pltpu.VMEM((16, 128), jnp.int32),
               pltpu.VMEM((8, 128), jnp.int32),
               pltpu.VMEM_SHARED((_NP, 128), jnp.float32),
               pltpu.SemaphoreType.DMA]
    return pl.kernel(_route_body, out_type=outs, scratch_types=scratch,
                     mesh=mesh)


def kernel(nodeInfosTensor, parent_index, W_resize, b_resize, W_parent,
           b_parent, W_nbr, b_nbr, W_ih, W_hh, b_ih, b_hh):
    pidx = parent_index.astype(jnp.int32)
    xf = jnp.pad(nodeInfosTensor,
                 ((0, 0), (0, _NP - _N), (0, 0))).reshape(_R, _D)
    sidxf = jnp.pad(
        jnp.pad(pidx, ((0, 0), (0, _NP - _N)),
                constant_values=_N).reshape(_B * 16, 5, 128),
        ((0, 0), (0, 3), (0, 0)), constant_values=_N).reshape(_B * 16 * 8, 128)
    gidxf = jnp.pad(
        (jnp.pad(pidx, ((0, 0), (0, _NP - _N)))
         + (jnp.arange(_B, dtype=jnp.int32) * _NP)[:, None]
         ).reshape(_NW, 10, 128),
        ((0, 0), (0, 6), (0, 0))).reshape(_NW * 16, 128)
    wcat = jnp.concatenate([W_parent, W_nbr, W_ih.T], axis=1)
    bcat = jnp.concatenate(
        [b_parent, jnp.zeros((128,), jnp.float32), b_ih]).reshape(1, 640)
    whh = W_hh.T
    bhh = b_hh.reshape(1, 384)
    bnbr = b_nbr.reshape(1, 128)
    br = b_resize.reshape(1, 128)
    zrow = jnp.zeros((128, 128), jnp.float32)
    onerow = jnp.ones((128, 128), jnp.float32)

    route = _get_route()

    def depth_body(h, _):
        hp, hn, gi = _stage_cat(h, wcat, bcat)
        p, sseg, cnt = route(hp, hn, gidxf, sidxf, zrow, onerow)
        h = _stage_gru(p, sseg, cnt, gi, whh, bhh, bnbr)
        return h, None

    h0 = _stage_resize(xf, W_resize, br)
    h, _ = lax.scan(depth_body, h0, None, length=3)
    return h.reshape(_B, _NP, _D)[:, :_N, :]


# speedup vs baseline: 1025.5614x; 1.2830x over previous
import functools

import jax
import jax.numpy as jnp
from jax import lax
from jax.experimental import pallas as pl
from jax.experimental.pallas import tpu as pltpu
from jax.experimental.pallas import tpu_sc as plsc

_B, _N, _D = 4, 10000, 128
_NP = 10240
_R = _B * _NP
_NW = 32
_PER_W = _R // _NW
_PER_S = _NP // 16
_BLK = 512
_NB = _R // _BLK


def _pre_body(x_ref, wr_ref, br_ref, wpn_ref, bpn_ref,
              h_ref, hp_ref, hn_ref):
    h = jnp.dot(x_ref[...], wr_ref[...],
                preferred_element_type=jnp.float32) + br_ref[...]
    h_ref[...] = h
    cat = jnp.dot(h, wpn_ref[...],
                  preferred_element_type=jnp.float32) + bpn_ref[...]
    hp_ref[...] = cat[:, :128]
    hn_ref[...] = cat[:, 128:]


def _fused_body(p_ref, s_ref, c_ref, h_ref, wih_ref, bih_ref, whh_ref,
                bhh_ref, bnbr_ref, wpn_ref, bpn_ref,
                ho_ref, hp_ref, hn_ref):
    h = h_ref[...]
    gi = jnp.dot(h, wih_ref[...],
                 preferred_element_type=jnp.float32) + bih_ref[...]
    inv = 1.0 / jnp.maximum(c_ref[...][:, :1], 1.0)
    summary = p_ref[...] + s_ref[...] * inv + bnbr_ref[...]
    gh = jnp.dot(summary, whh_ref[...],
                 preferred_element_type=jnp.float32) + bhh_ref[...]
    r = jax.nn.sigmoid(gi[:, :128] + gh[:, :128])
    z = jax.nn.sigmoid(gi[:, 128:256] + gh[:, 128:256])
    n = jnp.tanh(gi[:, 256:] + r * gh[:, 256:])
    hnew = (1.0 - z) * n + z * summary
    ho_ref[...] = hnew
    cat = jnp.dot(hnew, wpn_ref[...],
                  preferred_element_type=jnp.float32) + bpn_ref[...]
    hp_ref[...] = cat[:, :128]
    hn_ref[...] = cat[:, 128:]


def _row_spec(width):
    return pl.BlockSpec((_BLK, width), lambda i: (i, 0))


def _whole(shape):
    return pl.BlockSpec(shape, lambda i: tuple(0 for _ in shape))


_H_SHAPE = jax.ShapeDtypeStruct((_R, 128), jnp.float32)


def _stage_pre(xf, wr, br, wpn, bpn):
    return pl.pallas_call(
        _pre_body,
        grid=(_NB,),
        in_specs=[_row_spec(128), _whole((128, 128)), _whole((1, 128)),
                  _whole((128, 256)), _whole((1, 256))],
        out_specs=[_row_spec(128), _row_spec(128), _row_spec(128)],
        out_shape=[_H_SHAPE, _H_SHAPE, _H_SHAPE],
    )(xf, wr, br, wpn, bpn)


def _stage_fused(p, s, cnt, h, wih, bih, whh, bhh, bnbr, wpn, bpn):
    return pl.pallas_call(
        _fused_body,
        grid=(_NB,),
        in_specs=[_row_spec(128), _row_spec(128), _row_spec(128),
                  _row_spec(128), _whole((128, 384)), _whole((1, 384)),
                  _whole((128, 384)), _whole((1, 384)), _whole((1, 128)),
                  _whole((128, 256)), _whole((1, 256))],
        out_specs=[_row_spec(128), _row_spec(128), _row_spec(128)],
        out_shape=[_H_SHAPE, _H_SHAPE, _H_SHAPE],
    )(p, s, cnt, h, wih, bih, whh, bhh, bnbr, wpn, bpn)


def _route_body(hp, hn, gidxf, sidxf, zrow, onerow,
                p_out, s_out, c_out,
                rows_a, rows_b, gidxv, sidxv,
                acc, gsem, wsem, ssem):
    c = lax.axis_index("c")
    s = lax.axis_index("s")
    w = c * 16 + s
    bufs = (rows_a, rows_b)
    pltpu.sync_copy(gidxf.at[pl.ds(w * 16, 16)], gidxv)
    gd = {}
    wd = {}
    gd[0] = pltpu.async_copy(hp.at[gidxv.at[0]], bufs[0], gsem)
    for k in range(10):
        b = bufs[k % 2]
        gd[k].wait()
        wd[k] = pltpu.async_copy(
            b, p_out.at[pl.ds(w * _PER_W + k * 128, 128)], wsem)
        if k >= 1:
            wd[k - 1].wait()
        if k < 9:
            gd[k + 1] = pltpu.async_copy(
                hp.at[gidxv.at[k + 1]], bufs[(k + 1) % 2], gsem)
    wd[9].wait()
    for tr in range(2):
        t = 2 * c + tr
        pltpu.sync_copy(sidxf.at[pl.ds((t * 16 + s) * 8, 8)], sidxv)
        for use_ones in (False, True):
            out = c_out if use_ones else s_out
            pltpu.sync_copy(zrow, rows_a)
            zd = [pltpu.async_copy(
                rows_a, acc.at[pl.ds(s * _PER_S + i * 128, 128)], wsem)
                for i in range(5)]
            for d in zd:
                d.wait()
            plsc.subcore_barrier()
            if use_ones:
                pltpu.sync_copy(onerow, rows_a)
                sd = [pltpu.async_copy(rows_a, acc.at[sidxv.at[k]], ssem,
                                       add=True)
                      for k in range(5)]
                for d in sd:
                    d.wait()
            else:
                ld = {}
                sd = {}
                ld[0] = pltpu.async_copy(
                    hn.at[pl.ds(t * _NP + s * _PER_S, 128)], bufs[0], gsem)
                for k in range(5):
                    b = bufs[k % 2]
                    ld[k].wait()
                    sd[k] = pltpu.async_copy(b, acc.at[sidxv.at[k]], ssem,
                                             add=True)
                    if k >= 1:
                        sd[k - 1].wait()
                    if k < 4:
                        ld[k + 1] = pltpu.async_copy(
                            hn.at[pl.ds(t * _NP + s * _PER_S
                                        + (k + 1) * 128, 128)],
                            bufs[(k + 1) % 2], gsem)
                sd[4].wait()
            plsc.subcore_barrier()
            dd = {}
            od = {}
            dd[0] = pltpu.async_copy(
                acc.at[pl.ds(s * _PER_S, 128)], bufs[0], gsem)
            for i in range(5):
                b = bufs[i % 2]
                dd[i].wait()
                od[i] = pltpu.async_copy(
                    b, out.at[pl.ds(t * _NP + s * _PER_S + i * 128, 128)],
                    wsem)
                if i >= 1:
                    od[i - 1].wait()
                if i < 4:
                    dd[i + 1] = pltpu.async_copy(
                        acc.at[pl.ds(s * _PER_S + (i + 1) * 128, 128)],
                        bufs[(i + 1) % 2], gsem)
            od[4].wait()


@functools.lru_cache(maxsize=None)
def _get_route():
    mesh = plsc.VectorSubcoreMesh(core_axis_name="c", subcore_axis_name="s")
    outs = [jax.ShapeDtypeStruct((_R, 128), jnp.float32),
            jax.ShapeDtypeStruct((_R, 128), jnp.float32),
            jax.ShapeDtypeStruct((_R, 128), jnp.float32)]
    scratch = [pltpu.VMEM((128, 128), jnp.float32),
               pltpu.VMEM((128, 128), jnp.float32),
               pltpu.VMEM((16, 128), jnp.int32),
               pltpu.VMEM((8, 128), jnp.int32),
               pltpu.VMEM_SHARED((_NP, 128), jnp.float32),
               pltpu.SemaphoreType.DMA,
               pltpu.SemaphoreType.DMA,
               pltpu.SemaphoreType.DMA]
    return pl.kernel(_route_body, out_type=outs, scratch_types=scratch,
                     mesh=mesh)


def kernel(nodeInfosTensor, parent_index, W_resize, b_resize, W_parent,
           b_parent, W_nbr, b_nbr, W_ih, W_hh, b_ih, b_hh):
    pidx = parent_index.astype(jnp.int32)
    xf = jnp.pad(nodeInfosTensor,
                 ((0, 0), (0, _NP - _N), (0, 0))).reshape(_R, _D)
    sidxf = jnp.pad(
        jnp.pad(pidx, ((0, 0), (0, _NP - _N)),
                constant_values=_N).reshape(_B * 16, 5, 128),
        ((0, 0), (0, 3), (0, 0)), constant_values=_N).reshape(_B * 16 * 8, 128)
    gidxf = jnp.pad(
        (jnp.pad(pidx, ((0, 0), (0, _NP - _N)))
         + (jnp.arange(_B, dtype=jnp.int32) * _NP)[:, None]
         ).reshape(_NW, 10, 128),
        ((0, 0), (0, 6), (0, 0))).reshape(_NW * 16, 128)
    wpn = jnp.concatenate([W_parent, W_nbr], axis=1)
    bpn = jnp.concatenate(
        [b_parent, jnp.zeros((128,), jnp.float32)]).reshape(1, 256)
    wih = W_ih.T
    bih = b_ih.reshape(1, 384)
    whh = W_hh.T
    bhh = b_hh.reshape(1, 384)
    bnbr = b_nbr.reshape(1, 128)
    br = b_resize.reshape(1, 128)
    zrow = jnp.zeros((128, 128), jnp.float32)
    onerow = jnp.ones((128, 128), jnp.float32)

    route = _get_route()

    def depth_body(carry, _):
        h, hp, hn = carry
        p, sseg, cnt = route(hp, hn, gidxf, sidxf, zrow, onerow)
        h, hp, hn = _stage_fused(p, sseg, cnt, h, wih, bih, whh, bhh,
                                 bnbr, wpn, bpn)
        return (h, hp, hn), None

    h0, hp0, hn0 = _stage_pre(xf, W_resize, br, wpn, bpn)
    (h, _, _), _ = lax.scan(depth_body, (h0, hp0, hn0), None, length=3)
    return h.reshape(_B, _NP, _D)[:, :_N, :]
